# Initial kernel scaffold; baseline (speedup 1.0000x reference)
import jax
import jax.numpy as jnp
from jax.experimental import pallas as pl


def kernel(points, normals, enc_W1, enc_b1, enc_W2, enc_b2, enc_W3, enc_b3, dec_W1, dec_b1, dec_W2, dec_b2, dec_W3, dec_b3):
    raise NotImplementedError("write your pallas kernel here")


# speedup vs baseline: 8.4136x; 8.4136x over previous
import functools
import math

import jax
import jax.numpy as jnp
from jax import lax
from jax.experimental import pallas as pl
from jax.experimental.pallas import tpu as pltpu
from jax.experimental.pallas import tpu_sc as plsc

N_POINTS = 10000
NB = 32
RADIUS = 0.2
F1, F2, FOUT = 128, 256, 512
N_FPS = int(math.ceil(N_POINTS / NB))
E = N_FPS * NB
NPAD = 320
EPAD = NPAD * NB
TBL_W = 16
GCHUNK = 80
NW = 32
ROWS_PER_W = EPAD // NW


def _fps_body(points_ref, normals_ref, xyz_ref, out_p_ref, out_n_ref):
    X = xyz_ref[0:1, :]
    Y = xyz_ref[1:2, :]
    Z = xyz_ref[2:3, :]
    out_p_ref[0:1, :] = points_ref[0:1, :]
    out_n_ref[0:1, :] = normals_ref[0:1, :]
    q0 = points_ref[0:1, :]
    x0 = jnp.sum(q0[:, 0:1])
    y0 = jnp.sum(q0[:, 1:2])
    z0 = jnp.sum(q0[:, 2:3])
    dx = X - x0
    dy = Y - y0
    dz = Z - z0
    dists0 = dx * dx + dy * dy + dz * dz
    iota = lax.broadcasted_iota(jnp.int32, (1, N_POINTS), 1)

    def body(i, dists):
        m = jnp.max(dists)
        nxt = jnp.min(jnp.where(dists == m, iota, N_POINTS))
        row_p = points_ref[pl.ds(nxt, 1), :]
        row_n = normals_ref[pl.ds(nxt, 1), :]
        out_p_ref[pl.ds(i, 1), :] = row_p
        out_n_ref[pl.ds(i, 1), :] = row_n
        qx = jnp.sum(row_p[:, 0:1])
        qy = jnp.sum(row_p[:, 1:2])
        qz = jnp.sum(row_p[:, 2:3])
        ddx = X - qx
        ddy = Y - qy
        ddz = Z - qz
        d = ddx * ddx + ddy * ddy + ddz * ddz
        return jnp.minimum(dists, d)

    lax.fori_loop(1, N_FPS, body, dists0)


def _run_fps(points, normals, xyz):
    return pl.pallas_call(
        _fps_body,
        out_shape=[
            jax.ShapeDtypeStruct((N_FPS, 3), jnp.float32),
            jax.ShapeDtypeStruct((N_FPS, 3), jnp.float32),
        ],
    )(points, normals, xyz)


def _topk_body(fps_ref, xyz_ref, out_ref):
    fx = fps_ref[:, 0:1]
    fy = fps_ref[:, 1:2]
    fz = fps_ref[:, 2:3]
    X = xyz_ref[0:1, :]
    Y = xyz_ref[1:2, :]
    Z = xyz_ref[2:3, :]
    dx = fx - X
    dy = fy - Y
    dz = fz - Z
    D = dx * dx + dy * dy + dz * dz
    iota = lax.broadcasted_iota(jnp.int32, (8, N_POINTS), 1)
    inf = jnp.float32(jnp.inf)
    r2 = jnp.float32(RADIUS * RADIUS)
    first = None
    for k in range(NB):
        m = jnp.min(D, axis=1, keepdims=True)
        idx = jnp.min(jnp.where(D == m, iota, N_POINTS), axis=1,
                      keepdims=True)
        if k == 0:
            first = idx
            out_ref[:, 0:1] = idx
        else:
            out_ref[:, k:k + 1] = jnp.where(m <= r2, idx, first)
        D = jnp.where(iota == idx, inf, D)


def _run_topk(fps_pad, xyz):
    return pl.pallas_call(
        _topk_body,
        grid=(NPAD // 8,),
        in_specs=[
            pl.BlockSpec((8, 3), lambda i: (i, 0)),
            pl.BlockSpec((8, N_POINTS), lambda i: (0, 0)),
        ],
        out_specs=pl.BlockSpec((8, NB), lambda i: (i, 0)),
        out_shape=jax.ShapeDtypeStruct((NPAD, NB), jnp.int32),
    )(fps_pad, xyz)


def _sc_gather(table, idx2d):
    mesh = plsc.VectorSubcoreMesh(core_axis_name="c", subcore_axis_name="s")

    @functools.partial(
        pl.kernel,
        mesh=mesh,
        compiler_params=pltpu.CompilerParams(use_tc_tiling_on_sc=False),
        out_type=jax.ShapeDtypeStruct((EPAD, TBL_W), jnp.float32),
        scratch_types=[
            pltpu.VMEM((ROWS_PER_W // GCHUNK, GCHUNK), jnp.int32),
            pltpu.VMEM((ROWS_PER_W, TBL_W), jnp.float32),
            pltpu.SemaphoreType.DMA,
        ],
    )
    def gather_kernel(table_hbm, idx_hbm, out_hbm, idx_v, rows_v, sem):
        wid = lax.axis_index("s") * 2 + lax.axis_index("c")
        nchunk = ROWS_PER_W // GCHUNK
        pltpu.sync_copy(idx_hbm.at[pl.ds(wid * nchunk, nchunk)], idx_v)
        for j in range(nchunk):
            pltpu.async_copy(
                table_hbm.at[idx_v.at[j]],
                rows_v.at[pl.ds(j * GCHUNK, GCHUNK)],
                sem,
            ).wait()
        pltpu.sync_copy(rows_v, out_hbm.at[pl.ds(wid * ROWS_PER_W, ROWS_PER_W)])

    return gather_kernel(table, idx2d)


def _enc_body(g_ref, m_ref, w1_ref, b1_ref, w2_ref, b2_ref, w3_ref, b3_ref,
              enc_ref, ang_ref):
    g = g_ref[...]
    mm = m_ref[...]
    rad = g[:, 0:3]
    rn = g[:, 3:6]
    mid = mm[:, 0:3]
    mn = mm[:, 3:6]
    rel = (rad - mid) / jnp.float32(RADIUS)

    def dot(a, b):
        return jnp.sum(a * b, axis=1, keepdims=True)

    n_rel = dot(rel, rel)
    n_mn = dot(mn, mn)
    n_rn = dot(rn, rn)
    d1 = dot(mn, rel)
    d2 = dot(rn, rel)
    d3 = dot(mn, rn)
    eps = jnp.float32(1e-12)

    def angle(na, nb, d):
        c2 = jnp.maximum(na * nb - d * d, 0.0)
        return jnp.arctan2(jnp.sqrt(c2 + eps), d)

    a1 = angle(n_mn, n_rel, d1)
    a2 = angle(n_rn, n_rel, d2)
    a3 = angle(n_mn, n_rn, d3)
    norms = jnp.sqrt(n_rel + eps)
    inv = jnp.concatenate([a1, a2, a3, norms], axis=1)
    h = jnp.dot(inv, w1_ref[...], preferred_element_type=jnp.float32)
    h = jnp.maximum(h + b1_ref[...], 0.0)
    h = jnp.dot(h, w2_ref[...], preferred_element_type=jnp.float32)
    h = jnp.maximum(h + b2_ref[...], 0.0)
    h = jnp.dot(h, w3_ref[...], preferred_element_type=jnp.float32)
    h = h + b3_ref[...]
    eb = h.shape[0]
    enc_ref[...] = jnp.max(h.reshape(eb // NB, NB, FOUT), axis=1)
    ang_ref[...] = jnp.sum(rel.reshape(eb // NB, NB, 3), axis=1) / jnp.float32(NB)


def _run_enc(G, M, w1, b1, w2, b2, w3, b3):
    EB = 1024
    CB = EB // NB
    full = lambda shape: pl.BlockSpec(shape, lambda i: (0, 0))
    return pl.pallas_call(
        _enc_body,
        grid=(EPAD // EB,),
        in_specs=[
            pl.BlockSpec((EB, TBL_W), lambda i: (i, 0)),
            pl.BlockSpec((EB, TBL_W), lambda i: (i, 0)),
            full((4, F1)), full((1, F1)),
            full((F1, F2)), full((1, F2)),
            full((F2, FOUT)), full((1, FOUT)),
        ],
        out_specs=[
            pl.BlockSpec((CB, FOUT), lambda i: (i, 0)),
            pl.BlockSpec((CB, 3), lambda i: (i, 0)),
        ],
        out_shape=[
            jax.ShapeDtypeStruct((NPAD, FOUT), jnp.float32),
            jax.ShapeDtypeStruct((NPAD, 3), jnp.float32),
        ],
    )(G, M, w1, b1, w2, b2, w3, b3)


def _dec_body(ang_ref, enc_ref, fps96_ref, w1a_ref, w1b_ref, b1_ref,
              w2_ref, b2_ref, w3_ref, b3_ref, out_ref):
    g = jnp.dot(ang_ref[...], w1a_ref[...], preferred_element_type=jnp.float32)
    g = g + jnp.dot(enc_ref[...], w1b_ref[...],
                    preferred_element_type=jnp.float32)
    g = jnp.maximum(g + b1_ref[...], 0.0)
    g = jnp.maximum(
        jnp.dot(g, w2_ref[...], preferred_element_type=jnp.float32)
        + b2_ref[...], 0.0)
    g = jnp.dot(g, w3_ref[...], preferred_element_type=jnp.float32) + b3_ref[...]
    out_ref[...] = g * jnp.float32(RADIUS) + fps96_ref[...]


def _run_dec(ang, enc, fps96, w1a, w1b, b1, w2, b2, w3, b3):
    return pl.pallas_call(
        _dec_body,
        out_shape=jax.ShapeDtypeStruct((NPAD, NB * 3), jnp.float32),
    )(ang, enc, fps96, w1a, w1b, b1, w2, b2, w3, b3)


def kernel(points, normals, enc_W1, enc_b1, enc_W2, enc_b2, enc_W3, enc_b3,
           dec_W1, dec_b1, dec_W2, dec_b2, dec_W3, dec_b3):
    xyz = jnp.pad(points.T, ((0, 5), (0, 0)))
    fps_p, fps_n = _run_fps(points, normals, xyz)

    fps_pad = jnp.pad(fps_p, ((0, NPAD - N_FPS), (0, 0)),
                      constant_values=1e6)
    idx = _run_topk(fps_pad, xyz)
    rad_inds = idx[:N_FPS].reshape(-1)

    idx_flat = jnp.pad(rad_inds, (0, EPAD - E))
    idx2d = idx_flat.reshape(EPAD // GCHUNK, GCHUNK)
    table = jnp.pad(jnp.concatenate([points, normals], axis=1),
                    ((0, 0), (0, TBL_W - 6)))
    G = _sc_gather(table, idx2d)

    mid6 = jnp.concatenate([fps_p, fps_n], axis=1)
    M = jnp.broadcast_to(mid6[:, None, :], (N_FPS, NB, 6)).reshape(E, 6)
    M = jnp.pad(M, ((0, EPAD - E), (0, TBL_W - 6)))

    b = lambda v: v.reshape(1, -1)
    enc, ang = _run_enc(G, M, enc_W1, b(enc_b1), enc_W2, b(enc_b2),
                        enc_W3, b(enc_b3))

    fps96 = jnp.tile(fps_p, (1, NB))
    fps96 = jnp.pad(fps96, ((0, NPAD - N_FPS), (0, 0)))
    ang_pad = jnp.pad(ang[:N_FPS], ((0, NPAD - N_FPS), (0, 0)))
    enc_pad = jnp.pad(enc[:N_FPS], ((0, NPAD - N_FPS), (0, 0)))
    dec = _run_dec(ang_pad, enc_pad, fps96,
                   dec_W1[:3], dec_W1[3:], b(dec_b1),
                   dec_W2, b(dec_b2), dec_W3, b(dec_b3))

    rad_points = G[:E, 0:3]
    decoded_abs = dec[:N_FPS].reshape(E, 3)
    rad_cluster = jnp.broadcast_to(
        jnp.arange(N_FPS, dtype=jnp.int32)[:, None], (N_FPS, NB)).reshape(-1)
    return (rad_points, rad_cluster, decoded_abs, rad_cluster)


# speedup vs baseline: 15.1080x; 1.7957x over previous
import functools
import math

import jax
import jax.numpy as jnp
from jax import lax
from jax.experimental import pallas as pl
from jax.experimental.pallas import tpu as pltpu
from jax.experimental.pallas import tpu_sc as plsc

N_POINTS = 10000
NB = 32
RADIUS = 0.2
F1, F2, FOUT = 128, 256, 512
N_FPS = int(math.ceil(N_POINTS / NB))
E = N_FPS * NB
NPAD = 320
EPAD = NPAD * NB
TBL_W = 16
GCHUNK = 80
NW = 32
ROWS_PER_W = EPAD // NW


NROW = 80


def _fps_body(points_ref, normals_ref, grid_ref, out_p_ref, out_n_ref):
    X = grid_ref[0:NROW, :]
    Y = grid_ref[NROW:2 * NROW, :]
    Z = grid_ref[2 * NROW:3 * NROW, :]
    out_p_ref[0:1, :] = points_ref[0:1, :]
    out_n_ref[0:1, :] = normals_ref[0:1, :]
    q0 = points_ref[0:1, :]
    x0 = jnp.sum(q0[:, 0:1])
    y0 = jnp.sum(q0[:, 1:2])
    z0 = jnp.sum(q0[:, 2:3])
    dx = X - x0
    dy = Y - y0
    dz = Z - z0
    iota = (lax.broadcasted_iota(jnp.int32, (NROW, 128), 0) * 128
            + lax.broadcasted_iota(jnp.int32, (NROW, 128), 1))
    neg_inf = jnp.float32(-jnp.inf)
    dists0 = jnp.where(iota < N_POINTS, dx * dx + dy * dy + dz * dz, neg_inf)

    def body(i, dists):
        m = jnp.max(dists)
        nxt = jnp.min(jnp.where(dists == m, iota, N_POINTS))
        row_p = points_ref[pl.ds(nxt, 1), :]
        row_n = normals_ref[pl.ds(nxt, 1), :]
        out_p_ref[pl.ds(i, 1), :] = row_p
        out_n_ref[pl.ds(i, 1), :] = row_n
        qx = jnp.sum(row_p[:, 0:1])
        qy = jnp.sum(row_p[:, 1:2])
        qz = jnp.sum(row_p[:, 2:3])
        ddx = X - qx
        ddy = Y - qy
        ddz = Z - qz
        d = ddx * ddx + ddy * ddy + ddz * ddz
        return jnp.minimum(dists, d)

    lax.fori_loop(1, N_FPS, body, dists0)


def _run_fps(points, normals, grid):
    return pl.pallas_call(
        _fps_body,
        out_shape=[
            jax.ShapeDtypeStruct((N_FPS, 3), jnp.float32),
            jax.ShapeDtypeStruct((N_FPS, 3), jnp.float32),
        ],
    )(points, normals, grid)


TKR = 32


def _topk_body(fps_ref, xyz_ref, out_ref):
    fx = fps_ref[:, 0:1]
    fy = fps_ref[:, 1:2]
    fz = fps_ref[:, 2:3]
    X = xyz_ref[0:1, :]
    Y = xyz_ref[1:2, :]
    Z = xyz_ref[2:3, :]
    dx = fx - X
    dy = fy - Y
    dz = fz - Z
    D = dx * dx + dy * dy + dz * dz
    iota = lax.broadcasted_iota(jnp.int32, (TKR, N_POINTS), 1)
    inf = jnp.float32(jnp.inf)
    r2 = jnp.float32(RADIUS * RADIUS)
    first = None
    for k in range(NB):
        m = jnp.min(D, axis=1, keepdims=True)
        idx = jnp.min(jnp.where(D == m, iota, N_POINTS), axis=1,
                      keepdims=True)
        if k == 0:
            first = idx
            out_ref[:, 0:1] = idx
        else:
            out_ref[:, k:k + 1] = jnp.where(m <= r2, idx, first)
        D = jnp.where(iota == idx, inf, D)


def _run_topk(fps_pad, xyz):
    return pl.pallas_call(
        _topk_body,
        grid=(NPAD // TKR,),
        in_specs=[
            pl.BlockSpec((TKR, 3), lambda i: (i, 0)),
            pl.BlockSpec((8, N_POINTS), lambda i: (0, 0)),
        ],
        out_specs=pl.BlockSpec((TKR, NB), lambda i: (i, 0)),
        out_shape=jax.ShapeDtypeStruct((NPAD, NB), jnp.int32),
    )(fps_pad, xyz)


def _sc_gather(table, idx2d):
    mesh = plsc.VectorSubcoreMesh(core_axis_name="c", subcore_axis_name="s")

    @functools.partial(
        pl.kernel,
        mesh=mesh,
        compiler_params=pltpu.CompilerParams(use_tc_tiling_on_sc=False),
        out_type=jax.ShapeDtypeStruct((EPAD, TBL_W), jnp.float32),
        scratch_types=[
            pltpu.VMEM((ROWS_PER_W // GCHUNK, GCHUNK), jnp.int32),
            pltpu.VMEM((ROWS_PER_W, TBL_W), jnp.float32),
            pltpu.SemaphoreType.DMA,
        ],
    )
    def gather_kernel(table_hbm, idx_hbm, out_hbm, idx_v, rows_v, sem):
        wid = lax.axis_index("s") * 2 + lax.axis_index("c")
        nchunk = ROWS_PER_W // GCHUNK
        pltpu.sync_copy(idx_hbm.at[pl.ds(wid * nchunk, nchunk)], idx_v)
        for j in range(nchunk):
            pltpu.async_copy(
                table_hbm.at[idx_v.at[j]],
                rows_v.at[pl.ds(j * GCHUNK, GCHUNK)],
                sem,
            ).wait()
        pltpu.sync_copy(rows_v, out_hbm.at[pl.ds(wid * ROWS_PER_W, ROWS_PER_W)])

    return gather_kernel(table, idx2d)


def _enc_body(g_ref, m_ref, w1_ref, b1_ref, w2_ref, b2_ref, w3_ref, b3_ref,
              enc_ref, ang_ref):
    g = g_ref[...]
    mm = m_ref[...]
    rad = g[:, 0:3]
    rn = g[:, 3:6]
    mid = mm[:, 0:3]
    mn = mm[:, 3:6]
    rel = (rad - mid) / jnp.float32(RADIUS)

    def dot(a, b):
        return jnp.sum(a * b, axis=1, keepdims=True)

    n_rel = dot(rel, rel)
    n_mn = dot(mn, mn)
    n_rn = dot(rn, rn)
    d1 = dot(mn, rel)
    d2 = dot(rn, rel)
    d3 = dot(mn, rn)
    eps = jnp.float32(1e-12)

    def angle(na, nb, d):
        c2 = jnp.maximum(na * nb - d * d, 0.0)
        return jnp.arctan2(jnp.sqrt(c2 + eps), d)

    a1 = angle(n_mn, n_rel, d1)
    a2 = angle(n_rn, n_rel, d2)
    a3 = angle(n_mn, n_rn, d3)
    norms = jnp.sqrt(n_rel + eps)
    inv = jnp.concatenate([a1, a2, a3, norms], axis=1)
    h = jnp.dot(inv, w1_ref[...], preferred_element_type=jnp.float32)
    h = jnp.maximum(h + b1_ref[...], 0.0)
    h = jnp.dot(h, w2_ref[...], preferred_element_type=jnp.float32)
    h = jnp.maximum(h + b2_ref[...], 0.0)
    h = jnp.dot(h, w3_ref[...], preferred_element_type=jnp.float32)
    h = h + b3_ref[...]
    eb = h.shape[0]
    enc_ref[...] = jnp.max(h.reshape(eb // NB, NB, FOUT), axis=1)
    ang_ref[...] = jnp.sum(rel.reshape(eb // NB, NB, 3), axis=1) / jnp.float32(NB)


def _run_enc(G, M, w1, b1, w2, b2, w3, b3):
    EB = 1024
    CB = EB // NB
    full = lambda shape: pl.BlockSpec(shape, lambda i: (0, 0))
    return pl.pallas_call(
        _enc_body,
        grid=(EPAD // EB,),
        in_specs=[
            pl.BlockSpec((EB, TBL_W), lambda i: (i, 0)),
            pl.BlockSpec((EB, TBL_W), lambda i: (i, 0)),
            full((4, F1)), full((1, F1)),
            full((F1, F2)), full((1, F2)),
            full((F2, FOUT)), full((1, FOUT)),
        ],
        out_specs=[
            pl.BlockSpec((CB, FOUT), lambda i: (i, 0)),
            pl.BlockSpec((CB, 3), lambda i: (i, 0)),
        ],
        out_shape=[
            jax.ShapeDtypeStruct((NPAD, FOUT), jnp.float32),
            jax.ShapeDtypeStruct((NPAD, 3), jnp.float32),
        ],
    )(G, M, w1, b1, w2, b2, w3, b3)


def _dec_body(ang_ref, enc_ref, fps96_ref, w1a_ref, w1b_ref, b1_ref,
              w2_ref, b2_ref, w3_ref, b3_ref, out_ref):
    g = jnp.dot(ang_ref[...], w1a_ref[...], preferred_element_type=jnp.float32)
    g = g + jnp.dot(enc_ref[...], w1b_ref[...],
                    preferred_element_type=jnp.float32)
    g = jnp.maximum(g + b1_ref[...], 0.0)
    g = jnp.maximum(
        jnp.dot(g, w2_ref[...], preferred_element_type=jnp.float32)
        + b2_ref[...], 0.0)
    g = jnp.dot(g, w3_ref[...], preferred_element_type=jnp.float32) + b3_ref[...]
    out_ref[...] = g * jnp.float32(RADIUS) + fps96_ref[...]


def _run_dec(ang, enc, fps96, w1a, w1b, b1, w2, b2, w3, b3):
    return pl.pallas_call(
        _dec_body,
        out_shape=jax.ShapeDtypeStruct((NPAD, NB * 3), jnp.float32),
    )(ang, enc, fps96, w1a, w1b, b1, w2, b2, w3, b3)


def kernel(points, normals, enc_W1, enc_b1, enc_W2, enc_b2, enc_W3, enc_b3,
           dec_W1, dec_b1, dec_W2, dec_b2, dec_W3, dec_b3):
    xyz = jnp.pad(points.T, ((0, 5), (0, 0)))
    grid = jnp.pad(points.T, ((0, 0), (0, NROW * 128 - N_POINTS)),
                   constant_values=1e9).reshape(3 * NROW, 128)
    fps_p, fps_n = _run_fps(points, normals, grid)

    fps_pad = jnp.pad(fps_p, ((0, NPAD - N_FPS), (0, 0)),
                      constant_values=1e6)
    idx = _run_topk(fps_pad, xyz)
    rad_inds = idx[:N_FPS].reshape(-1)

    idx_flat = jnp.pad(rad_inds, (0, EPAD - E))
    idx2d = idx_flat.reshape(EPAD // GCHUNK, GCHUNK)
    table = jnp.pad(jnp.concatenate([points, normals], axis=1),
                    ((0, 0), (0, TBL_W - 6)))
    G = _sc_gather(table, idx2d)

    mid6 = jnp.concatenate([fps_p, fps_n], axis=1)
    M = jnp.broadcast_to(mid6[:, None, :], (N_FPS, NB, 6)).reshape(E, 6)
    M = jnp.pad(M, ((0, EPAD - E), (0, TBL_W - 6)))

    b = lambda v: v.reshape(1, -1)
    enc, ang = _run_enc(G, M, enc_W1, b(enc_b1), enc_W2, b(enc_b2),
                        enc_W3, b(enc_b3))

    fps96 = jnp.tile(fps_p, (1, NB))
    fps96 = jnp.pad(fps96, ((0, NPAD - N_FPS), (0, 0)))
    ang_pad = jnp.pad(ang[:N_FPS], ((0, NPAD - N_FPS), (0, 0)))
    enc_pad = jnp.pad(enc[:N_FPS], ((0, NPAD - N_FPS), (0, 0)))
    dec = _run_dec(ang_pad, enc_pad, fps96,
                   dec_W1[:3], dec_W1[3:], b(dec_b1),
                   dec_W2, b(dec_b2), dec_W3, b(dec_b3))

    rad_points = G[:E, 0:3]
    decoded_abs = dec[:N_FPS].reshape(E, 3)
    rad_cluster = jnp.broadcast_to(
        jnp.arange(N_FPS, dtype=jnp.int32)[:, None], (N_FPS, NB)).reshape(-1)
    return (rad_points, rad_cluster, decoded_abs, rad_cluster)
